# Initial kernel scaffold; baseline (speedup 1.0000x reference)
import jax
import jax.numpy as jnp
from jax.experimental import pallas as pl


def kernel(logits):
    raise NotImplementedError("write your pallas kernel here")


# speedup vs baseline: 6.8249x; 6.8249x over previous
import numpy as np

import jax
import jax.numpy as jnp
from jax.experimental import pallas as pl

_T = 0.5
_K = 16
_B = 64
_N = 32768

_EPS = np.float32(np.finfo(np.float32).eps)
_ONE_MINUS_EPS = np.float32(1.0) - _EPS


def _threefry_fold_key():
    def rotl(x, r):
        return ((x << r) | (x >> (32 - r))) & 0xFFFFFFFF

    def tf(k0, k1, x0, x1):
        ks2 = k0 ^ k1 ^ 0x1BD11BDA
        x0 = (x0 + k0) & 0xFFFFFFFF
        x1 = (x1 + k1) & 0xFFFFFFFF
        rots = ((13, 15, 26, 6), (17, 29, 16, 24))
        sched = ((k1, ks2, 1), (ks2, k0, 2), (k0, k1, 3), (k1, ks2, 4), (ks2, k0, 5))
        for i, (a, b, c) in enumerate(sched):
            for r in rots[i % 2]:
                x0 = (x0 + x1) & 0xFFFFFFFF
                x1 = rotl(x1, r) ^ x0
            x0 = (x0 + a) & 0xFFFFFFFF
            x1 = (x1 + b + c) & 0xFFFFFFFF
        return x0, x1

    return tf(0, 0, 0, 1)


_FK0, _FK1 = _threefry_fold_key()


def _threefry_bits(x1):
    k0 = jnp.uint32(_FK0)
    k1 = jnp.uint32(_FK1)
    ks2 = jnp.uint32(_FK0 ^ _FK1 ^ 0x1BD11BDA)

    def rotl(x, r):
        return (x << jnp.uint32(r)) | (x >> jnp.uint32(32 - r))

    x0 = jnp.zeros_like(x1) + k0
    x1 = x1 + k1
    rots = ((13, 15, 26, 6), (17, 29, 16, 24))
    sched = ((k1, ks2, 1), (ks2, k0, 2), (k0, k1, 3), (k1, ks2, 4), (ks2, k0, 5))
    for i, (a, b, c) in enumerate(sched):
        for r in rots[i % 2]:
            x0 = x0 + x1
            x1 = rotl(x1, r) ^ x0
        x0 = x0 + a
        x1 = x1 + b + jnp.uint32(c)
    return x0 ^ x1


def _body(logits_ref, ds_ref, cs_ref):
    b = pl.program_id(0)
    l = logits_ref[0]

    row = jax.lax.broadcasted_iota(jnp.int32, (_K, _N), 0)
    col = jax.lax.broadcasted_iota(jnp.int32, (_K, _N), 1)
    flat = (b * _K + row) * _N + col
    bits = _threefry_bits(flat.astype(jnp.uint32))
    fbits = (bits >> jnp.uint32(9)) | jnp.uint32(0x3F800000)
    u = jax.lax.bitcast_convert_type(fbits, jnp.float32) - jnp.float32(1.0)
    u = jnp.clip(jnp.maximum(u, jnp.float32(0.0)), _EPS, _ONE_MINUS_EPS)
    g = -jnp.log(-jnp.log(u))
    x = (g + l) * jnp.float32(1.0 / _T)
    m = jnp.max(x, axis=1, keepdims=True)
    e = jnp.exp(x - m)
    s = jnp.sum(e, axis=1, keepdims=True)
    cs_ref[0] = jnp.max(e / s, axis=0, keepdims=True)

    def step(_, carry):
        thr, removed, act = carry
        mx = jnp.max(act)
        cnt = jnp.sum(jnp.where(act == mx, jnp.float32(1.0), jnp.float32(0.0)))
        thr = jnp.where(removed < jnp.float32(_K), mx, thr)
        act = jnp.where(act == mx, -jnp.inf, act)
        return thr, removed + cnt, act

    thr, _, _ = jax.lax.fori_loop(
        0, _K, step, (jnp.float32(0.0), jnp.float32(0.0), l)
    )
    ds_ref[0] = jnp.where(l >= thr, jnp.float32(1.0), jnp.float32(0.0))


def kernel(logits):
    ds, cs = pl.pallas_call(
        _body,
        grid=(_B,),
        in_specs=[pl.BlockSpec((1, 1, _N), lambda b: (b, 0, 0))],
        out_specs=[
            pl.BlockSpec((1, 1, _N), lambda b: (b, 0, 0)),
            pl.BlockSpec((1, 1, _N), lambda b: (b, 0, 0)),
        ],
        out_shape=[
            jax.ShapeDtypeStruct((_B, 1, _N), jnp.float32),
            jax.ShapeDtypeStruct((_B, 1, _N), jnp.float32),
        ],
    )(logits)
    return ds.reshape(_B, _N), cs.reshape(_B, _N)


# speedup vs baseline: 8.4357x; 1.2360x over previous
import numpy as np

import jax
import jax.numpy as jnp
from jax.experimental import pallas as pl
from jax.experimental.pallas import tpu as pltpu

_T = 0.5
_K = 16
_B = 64
_N = 32768
_C = 1024
_NC = _N // _C

_EPS = np.float32(np.finfo(np.float32).eps)


def _threefry_fold_key():
    def rotl(x, r):
        return ((x << r) | (x >> (32 - r))) & 0xFFFFFFFF

    def tf(k0, k1, x0, x1):
        ks2 = k0 ^ k1 ^ 0x1BD11BDA
        x0 = (x0 + k0) & 0xFFFFFFFF
        x1 = (x1 + k1) & 0xFFFFFFFF
        rots = ((13, 15, 26, 6), (17, 29, 16, 24))
        sched = ((k1, ks2, 1), (ks2, k0, 2), (k0, k1, 3), (k1, ks2, 4), (ks2, k0, 5))
        for i, (a, b, c) in enumerate(sched):
            for r in rots[i % 2]:
                x0 = (x0 + x1) & 0xFFFFFFFF
                x1 = rotl(x1, r) ^ x0
            x0 = (x0 + a) & 0xFFFFFFFF
            x1 = (x1 + b + c) & 0xFFFFFFFF
        return x0, x1

    return tf(0, 0, 0, 1)


_FK0, _FK1 = _threefry_fold_key()


def _threefry_bits(x1):
    k0 = jnp.uint32(_FK0)
    k1 = jnp.uint32(_FK1)
    ks2 = jnp.uint32(_FK0 ^ _FK1 ^ 0x1BD11BDA)

    def rotl(x, r):
        return (x << jnp.uint32(r)) | (x >> jnp.uint32(32 - r))

    x0 = jnp.zeros_like(x1) + k0
    x1 = x1 + k1
    rots = ((13, 15, 26, 6), (17, 29, 16, 24))
    sched = ((k1, ks2, 1), (ks2, k0, 2), (k0, k1, 3), (k1, ks2, 4), (ks2, k0, 5))
    for i, (a, b, c) in enumerate(sched):
        for r in rots[i % 2]:
            x0 = x0 + x1
            x1 = rotl(x1, r) ^ x0
        x0 = x0 + a
        x1 = x1 + b + jnp.uint32(c)
    return x0 ^ x1


def _body(logits_ref, ds_ref, cs_ref, r_ref, e_ref):
    b = pl.program_id(0)
    l = logits_ref[0]

    lmax = jnp.max(l)
    e_ref[...] = jnp.exp((l - lmax) * jnp.float32(2.0 / _T * 0.5))

    row = jax.lax.broadcasted_iota(jnp.int32, (_K, _C), 0)
    col = jax.lax.broadcasted_iota(jnp.int32, (_K, _C), 1)
    base = (b * _K + row) * _N + col

    def pass_a(c, acc):
        n0 = c * _C
        idx = (base + n0).astype(jnp.uint32)
        bits = _threefry_bits(idx)
        fbits = (bits >> jnp.uint32(9)) | jnp.uint32(0x3F800000)
        u = jax.lax.bitcast_convert_type(fbits, jnp.float32) - jnp.float32(1.0)
        w = jnp.log(jnp.maximum(u, _EPS))
        r = jnp.float32(1.0) / (w * w)
        r_ref[:, pl.ds(n0, _C)] = r
        e = e_ref[0, pl.ds(n0, _C)]
        return acc + e[None, :] * r

    acc = jax.lax.fori_loop(0, _NC, pass_a, jnp.zeros((_K, _C), jnp.float32))
    inv_s = jnp.float32(1.0) / jnp.sum(acc, axis=1, keepdims=True)

    def pass_b(c, carry):
        n0 = c * _C
        r = r_ref[:, pl.ds(n0, _C)]
        t = jnp.max(r * inv_s, axis=0, keepdims=True)
        cs_ref[0, 0, pl.ds(n0, _C)] = e_ref[0, pl.ds(n0, _C)] * t[0]
        return carry

    jax.lax.fori_loop(0, _NC, pass_b, 0)

    def step(_, carry):
        thr, removed, act = carry
        mx = jnp.max(act)
        cnt = jnp.sum(jnp.where(act == mx, jnp.float32(1.0), jnp.float32(0.0)))
        thr = jnp.where(removed < jnp.float32(_K), mx, thr)
        act = jnp.where(act == mx, -jnp.inf, act)
        return thr, removed + cnt, act

    thr, _, _ = jax.lax.fori_loop(
        0, _K, step, (jnp.float32(0.0), jnp.float32(0.0), l)
    )
    ds_ref[0] = jnp.where(l >= thr, jnp.float32(1.0), jnp.float32(0.0))


def kernel(logits):
    ds, cs = pl.pallas_call(
        _body,
        grid=(_B,),
        in_specs=[pl.BlockSpec((1, 1, _N), lambda b: (b, 0, 0))],
        out_specs=[
            pl.BlockSpec((1, 1, _N), lambda b: (b, 0, 0)),
            pl.BlockSpec((1, 1, _N), lambda b: (b, 0, 0)),
        ],
        out_shape=[
            jax.ShapeDtypeStruct((_B, 1, _N), jnp.float32),
            jax.ShapeDtypeStruct((_B, 1, _N), jnp.float32),
        ],
        scratch_shapes=[
            pltpu.VMEM((_K, _N), jnp.float32),
            pltpu.VMEM((1, _N), jnp.float32),
        ],
    )(logits)
    return ds.reshape(_B, _N), cs.reshape(_B, _N)


# speedup vs baseline: 9.4331x; 1.1182x over previous
import numpy as np

import jax
import jax.numpy as jnp
from jax.experimental import pallas as pl
from jax.experimental.pallas import tpu as pltpu

_T = 0.5
_K = 16
_B = 64
_N = 32768
_S = 8
_L = _N // _S

_EPS = np.float32(np.finfo(np.float32).eps)


def _threefry_fold_key():
    def rotl(x, r):
        return ((x << r) | (x >> (32 - r))) & 0xFFFFFFFF

    def tf(k0, k1, x0, x1):
        ks2 = k0 ^ k1 ^ 0x1BD11BDA
        x0 = (x0 + k0) & 0xFFFFFFFF
        x1 = (x1 + k1) & 0xFFFFFFFF
        rots = ((13, 15, 26, 6), (17, 29, 16, 24))
        sched = ((k1, ks2, 1), (ks2, k0, 2), (k0, k1, 3), (k1, ks2, 4), (ks2, k0, 5))
        for i, (a, b, c) in enumerate(sched):
            for r in rots[i % 2]:
                x0 = (x0 + x1) & 0xFFFFFFFF
                x1 = rotl(x1, r) ^ x0
            x0 = (x0 + a) & 0xFFFFFFFF
            x1 = (x1 + b + c) & 0xFFFFFFFF
        return x0, x1

    return tf(0, 0, 0, 1)


_FK0, _FK1 = _threefry_fold_key()


def _threefry_bits(x1):
    k0 = jnp.uint32(_FK0)
    k1 = jnp.uint32(_FK1)
    ks2 = jnp.uint32(_FK0 ^ _FK1 ^ 0x1BD11BDA)

    def rotl(x, r):
        return (x << jnp.uint32(r)) | (x >> jnp.uint32(32 - r))

    x0 = jnp.zeros_like(x1) + k0
    x1 = x1 + k1
    rots = ((13, 15, 26, 6), (17, 29, 16, 24))
    sched = ((k1, ks2, 1), (ks2, k0, 2), (k0, k1, 3), (k1, ks2, 4), (ks2, k0, 5))
    for i, (a, b, c) in enumerate(sched):
        for r in rots[i % 2]:
            x0 = x0 + x1
            x1 = rotl(x1, r) ^ x0
        x0 = x0 + a
        x1 = x1 + b + jnp.uint32(c)
    return x0 ^ x1


def _body(logits_ref, ds_ref, cs_ref, r_ref, s_ref):
    b = pl.program_id(0)
    l = logits_ref[0]

    lmax = jnp.max(l)
    e = jnp.exp((l - lmax) * jnp.float32(2.0))

    base = (
        jax.lax.broadcasted_iota(jnp.int32, (_S, _L), 0) * _L
        + jax.lax.broadcasted_iota(jnp.int32, (_S, _L), 1)
        + b * (_K * _N)
    )

    def pass_a(k, _):
        idx = (base + k * _N).astype(jnp.uint32)
        bits = _threefry_bits(idx)
        fbits = (bits >> jnp.uint32(9)) | jnp.uint32(0x3F800000)
        u = jax.lax.bitcast_convert_type(fbits, jnp.float32) - jnp.float32(1.0)
        w = jnp.log(jnp.maximum(u, _EPS))
        r = jnp.float32(1.0) / (w * w)
        r_ref[k] = r
        s_ref[k, 0] = jnp.sum(e * r)
        return 0

    jax.lax.fori_loop(0, _K, pass_a, 0)

    def pass_b(k, best):
        return jnp.maximum(best, r_ref[k] * (jnp.float32(1.0) / s_ref[k, 0]))

    best = jax.lax.fori_loop(
        0, _K, pass_b, jnp.zeros((_S, _L), jnp.float32)
    )
    cs_ref[0] = e * best

    def step(_, carry):
        thr, removed, act = carry
        mx = jnp.max(act)
        cnt = jnp.sum(jnp.where(act == mx, jnp.float32(1.0), jnp.float32(0.0)))
        thr = jnp.where(removed < jnp.float32(_K), mx, thr)
        act = jnp.where(act == mx, -jnp.inf, act)
        return thr, removed + cnt, act

    thr, _, _ = jax.lax.fori_loop(
        0, _K, step, (jnp.float32(0.0), jnp.float32(0.0), l)
    )
    ds_ref[0] = jnp.where(l >= thr, jnp.float32(1.0), jnp.float32(0.0))


def kernel(logits):
    lg = logits.reshape(_B, _S, _L)
    ds, cs = pl.pallas_call(
        _body,
        grid=(_B,),
        in_specs=[pl.BlockSpec((1, _S, _L), lambda b: (b, 0, 0))],
        out_specs=[
            pl.BlockSpec((1, _S, _L), lambda b: (b, 0, 0)),
            pl.BlockSpec((1, _S, _L), lambda b: (b, 0, 0)),
        ],
        out_shape=[
            jax.ShapeDtypeStruct((_B, _S, _L), jnp.float32),
            jax.ShapeDtypeStruct((_B, _S, _L), jnp.float32),
        ],
        scratch_shapes=[
            pltpu.VMEM((_K, _S, _L), jnp.float32),
            pltpu.SMEM((_K, 1), jnp.float32),
        ],
    )(lg)
    return ds.reshape(_B, _N), cs.reshape(_B, _N)
